# baseline (speedup 1.0000x reference)
import jax
import jax.numpy as jnp
from jax import lax
from jax.experimental import pallas as pl
from jax.experimental.pallas import tpu as pltpu
from jax.experimental.pallas import tpu_sc as plsc

B, E, H, TOPK = 4096, 64, 256, 8
L = 16
NC, NS = 2, 16
NW = NC * NS
TPW = B // NW
EG = E // L

TB = 32
RB = TB * E

SZ_Q = TPW * E * 3
SZ_S = TPW * E
SZ_A = TPW * 3

_BF = jnp.bfloat16
_F32 = jnp.float32


def _dyng(x, idx):
    return jnp.take_along_axis(x, idx, axis=0, mode="promise_in_bounds")


def _score_body(xq_ref, xr_ref, xk_ref, xc_ref, wet_ref, wct_ref,
                be_ref, bc_ref, wse_ref, wsc_ref, bs_ref,
                s1_ref, cv_ref):
    ei16 = jnp.concatenate(
        [xq_ref[...].astype(_BF), xr_ref[...].astype(_BF),
         xk_ref[...].astype(_BF)], axis=1)
    ee = lax.dot_general(ei16, wet_ref[...].astype(_BF),
                         (((1,), (0,)), ((), ())),
                         preferred_element_type=_F32) + be_ref[...]
    s1_ref[...] = lax.dot_general(ee.astype(_BF), wse_ref[...].astype(_BF),
                                  (((1,), (0,)), ((), ())),
                                  preferred_element_type=_F32)
    ce = lax.dot_general(xc_ref[...].astype(_BF), wct_ref[...].astype(_BF),
                         (((1,), (0,)), ((), ())),
                         preferred_element_type=_F32) + bc_ref[...]
    cv_ref[...] = lax.dot_general(ce.astype(_BF), wsc_ref[...].astype(_BF),
                                  (((1,), (0,)), ((), ())),
                                  preferred_element_type=_F32) + bs_ref[...]


_score = pl.pallas_call(
    _score_body,
    grid=(B // TB,),
    in_specs=[
        pl.BlockSpec((RB, 3), lambda i: (i, 0)),
        pl.BlockSpec((RB, 2), lambda i: (i, 0)),
        pl.BlockSpec((RB, 2), lambda i: (i, 0)),
        pl.BlockSpec((TB, 68), lambda i: (i, 0)),
        pl.BlockSpec((7, H), lambda i: (0, 0)),
        pl.BlockSpec((68, H), lambda i: (0, 0)),
        pl.BlockSpec((1, H), lambda i: (0, 0)),
        pl.BlockSpec((1, H), lambda i: (0, 0)),
        pl.BlockSpec((H, 1), lambda i: (0, 0)),
        pl.BlockSpec((H, 1), lambda i: (0, 0)),
        pl.BlockSpec((1, 1), lambda i: (0, 0)),
    ],
    out_specs=[
        pl.BlockSpec((RB, 1), lambda i: (i, 0)),
        pl.BlockSpec((TB, 1), lambda i: (i, 0)),
    ],
    out_shape=[
        jax.ShapeDtypeStruct((B * E, 1), _F32),
        jax.ShapeDtypeStruct((B, 1), _F32),
    ],
)


def _sc_body(s1_h, cv_h, xq_h,
             act_h, gate_h, lg_h,
             s1_v, cv_v, xq_v, act_v, gate_v, lg_v):
    wid = lax.axis_index("s") * NC + lax.axis_index("c")

    pltpu.sync_copy(s1_h.at[pl.ds(wid * SZ_S, SZ_S)], s1_v)
    pltpu.sync_copy(cv_h.at[pl.ds(wid * TPW, TPW)], cv_v)
    pltpu.sync_copy(xq_h.at[pl.ds(wid * SZ_Q, SZ_Q)], xq_v)

    iota = lax.iota(jnp.int32, L)
    lt8 = iota < 8
    shift8 = (iota + 8) & 15
    zeros16 = jnp.zeros((L,), jnp.float32)

    def merge8(ak, av, bk, bv):
        bks, bvs = _dyng(bk, shift8), _dyng(bv, shift8)
        return jnp.where(lt8, ak, bks), jnp.where(lt8, av, bvs)

    def token(b, carry):
        cb = plsc.load_gather(cv_v, [jnp.zeros((L,), jnp.int32) + b])
        ks, vs = [], []
        for g in range(EG):
            e16 = g * L + iota
            lg = plsc.load_gather(s1_v, [b * E + e16]) + cb
            plsc.store_scatter(lg_v, [b * E + e16], lg)
            plsc.store_scatter(gate_v, [b * E + e16], zeros16)
            sk, sv = plsc.sort_key_val(lg, e16, descending=True)
            ks.append(sk)
            vs.append(sv)

        m0k, m0v = merge8(ks[0], vs[0], ks[1], vs[1])
        m1k, m1v = merge8(ks[2], vs[2], ks[3], vs[3])
        s0k, s0v = plsc.sort_key_val(m0k, m0v, descending=True)
        s1k, s1v = plsc.sort_key_val(m1k, m1v, descending=True)
        fk, fv = merge8(s0k, s0v, s1k, s1v)
        fk, fv = plsc.sort_key_val(fk, fv, descending=True)

        mx = _dyng(fk, jnp.zeros((L,), jnp.int32))
        ex = jnp.where(lt8, jnp.exp(fk - mx), 0.0)
        g8 = ex / jnp.sum(ex)
        plsc.store_scatter(gate_v, [b * E + fv], g8, mask=lt8)

        qb = b * (E * 3)
        a0 = jnp.sum(g8 * plsc.load_gather(xq_v, [qb + fv * 3]))
        a1 = jnp.sum(g8 * plsc.load_gather(xq_v, [qb + fv * 3 + 1]))
        a2 = jnp.sum(g8 * plsc.load_gather(xq_v, [qb + fv * 3 + 2]))
        outv = jnp.where(iota == 0, a0, jnp.where(iota == 1, a1, a2))
        plsc.store_scatter(act_v, [b * 3 + iota], outv, mask=iota < 3)
        return carry

    lax.fori_loop(0, TPW, token, 0)

    pltpu.sync_copy(act_v, act_h.at[pl.ds(wid * SZ_A, SZ_A)])
    pltpu.sync_copy(gate_v, gate_h.at[pl.ds(wid * SZ_S, SZ_S)])
    pltpu.sync_copy(lg_v, lg_h.at[pl.ds(wid * SZ_S, SZ_S)])


_sc_router = pl.kernel(
    _sc_body,
    out_type=[
        jax.ShapeDtypeStruct((B * 3,), jnp.float32),
        jax.ShapeDtypeStruct((B * E,), jnp.float32),
        jax.ShapeDtypeStruct((B * E,), jnp.float32),
    ],
    mesh=plsc.VectorSubcoreMesh(core_axis_name="c", subcore_axis_name="s"),
    compiler_params=pltpu.CompilerParams(needs_layout_passes=False),
    scratch_types=[
        pltpu.VMEM((SZ_S,), jnp.float32),
        pltpu.VMEM((TPW,), jnp.float32),
        pltpu.VMEM((SZ_Q,), jnp.float32),
        pltpu.VMEM((SZ_A,), jnp.float32),
        pltpu.VMEM((SZ_S,), jnp.float32),
        pltpu.VMEM((SZ_S,), jnp.float32),
    ],
)


def kernel(x_context, x_q_values, x_reward, x_risk, Wc, bc, We, be, Ws, bs):
    assert x_q_values.shape == (B, E, 3) and x_context.shape == (B, 68)
    s1, cv = _score(
        x_q_values.reshape(B * E, 3), x_reward.reshape(B * E, 2),
        x_risk.reshape(B * E, 2), x_context,
        We.T, Wc.T, be.reshape(1, H), bc.reshape(1, H),
        Ws[:, :H].reshape(H, 1), Ws[:, H:].reshape(H, 1), bs.reshape(1, 1))
    act, gate, lg = _sc_router(s1.reshape(-1), cv.reshape(-1),
                               x_q_values.reshape(-1))
    return act.reshape(B, 3), gate.reshape(B, E), lg.reshape(B, E)
